# Initial kernel scaffold; baseline (speedup 1.0000x reference)
import jax
import jax.numpy as jnp
from jax.experimental import pallas as pl


def kernel(x, gamma, beta, wq, wkv, wo, w1, b1, w2, b2):
    raise NotImplementedError("write your pallas kernel here")


# speedup vs baseline: 3.0858x; 3.0858x over previous
import functools

import jax
import jax.numpy as jnp
from jax.experimental import pallas as pl
from jax.experimental.pallas import tpu as pltpu


def _fused_block_kernel(x_ref, g_ref, beta_ref,
                        wq_ref, wk_ref, wv_ref, wo_ref,
                        w1x_ref, w1g_ref, b1x_ref, b1g_ref, w2_ref, b2_ref,
                        o_ref, *, heads, scale):
    x = x_ref[...].astype(jnp.float32)
    mu = jnp.mean(x, axis=-1, keepdims=True)
    var = jnp.mean(jnp.square(x - mu), axis=-1, keepdims=True)
    xn = (x - mu) * jax.lax.rsqrt(var + 1e-5)
    xn = xn * g_ref[...].astype(jnp.float32) + beta_ref[...].astype(jnp.float32)
    xnb = xn.astype(jnp.bfloat16)

    q = (jnp.dot(xnb, wq_ref[...], preferred_element_type=jnp.float32)
         * scale).astype(jnp.bfloat16)
    k = jnp.dot(xnb, wk_ref[...],
                preferred_element_type=jnp.float32).astype(jnp.bfloat16)
    v = jnp.dot(xnb, wv_ref[...],
                preferred_element_type=jnp.float32).astype(jnp.bfloat16)

    hx = jnp.dot(xnb, w1x_ref[...],
                 preferred_element_type=jnp.float32) + b1x_ref[...]
    hg = jnp.dot(xnb, w1g_ref[...],
                 preferred_element_type=jnp.float32) + b1g_ref[...]
    sw = (hx * (hg * jax.nn.sigmoid(hg))).astype(jnp.bfloat16)
    acc = jnp.dot(sw, w2_ref[...],
                  preferred_element_type=jnp.float32) + b2_ref[...]

    dh = q.shape[1] // heads
    for h in range(heads):
        sl = slice(h * dh, (h + 1) * dh)
        qh, kh, vh = q[:, sl], k[:, sl], v[:, sl]
        s = jax.lax.dot_general(qh, kh, (((1,), (1,)), ((), ())),
                                preferred_element_type=jnp.float32)
        m = jnp.max(s, axis=-1, keepdims=True)
        p = jnp.exp(s - m)
        inv_l = pl.reciprocal(jnp.sum(p, axis=-1, keepdims=True), approx=True)
        oh = jnp.dot(p.astype(jnp.bfloat16), vh,
                     preferred_element_type=jnp.float32) * inv_l
        acc = acc + jnp.dot(oh.astype(jnp.bfloat16), wo_ref[sl, :],
                            preferred_element_type=jnp.float32)

    o_ref[...] = acc.astype(o_ref.dtype)


def kernel(x, gamma, beta, wq, wkv, wo, w1, b1, w2, b2):
    B, N, D = x.shape
    heads = 4
    inner = wq.shape[1]
    ffd = w2.shape[0]
    scale = (inner // heads) ** -0.5
    bf = jnp.bfloat16

    wk_, wv_ = wkv[:, :inner], wkv[:, inner:]
    w1x, w1g = w1[:, :ffd], w1[:, ffd:]
    b1x, b1g = b1[:, :ffd], b1[:, ffd:]

    bmap = lambda b: (b, 0, 0)
    wmap = lambda b: (0, 0)
    full2 = lambda shape: pl.BlockSpec(shape, wmap)

    return pl.pallas_call(
        functools.partial(_fused_block_kernel, heads=heads, scale=scale),
        out_shape=jax.ShapeDtypeStruct((B, N, D), x.dtype),
        grid_spec=pltpu.PrefetchScalarGridSpec(
            num_scalar_prefetch=0,
            grid=(B,),
            in_specs=[
                pl.BlockSpec((pl.Squeezed(), N, D), bmap),
                full2((1, D)), full2((1, D)),
                full2((D, inner)), full2((D, inner)),
                full2((D, inner)), full2((inner, D)),
                full2((D, ffd)), full2((D, ffd)),
                full2((1, ffd)), full2((1, ffd)),
                full2((ffd, D)), full2((1, D)),
            ],
            out_specs=pl.BlockSpec((pl.Squeezed(), N, D), bmap),
        ),
        compiler_params=pltpu.CompilerParams(
            dimension_semantics=("parallel",),
            vmem_limit_bytes=56 * 1024 * 1024),
    )(x, gamma, beta, wq.astype(bf), wk_.astype(bf), wv_.astype(bf),
      wo.astype(bf), w1x.astype(bf), w1g.astype(bf), b1x, b1g,
      w2.astype(bf), b2)


# speedup vs baseline: 3.1213x; 1.0115x over previous
import functools

import jax
import jax.numpy as jnp
from jax.experimental import pallas as pl
from jax.experimental.pallas import tpu as pltpu


def _fused_block_kernel(x_ref, g_ref, beta_ref,
                        wqkv_ref, wo_ref, w1_ref, b1_ref, w2_ref, b2_ref,
                        o_ref, *, heads, inner, ffd):
    x = x_ref[...].astype(jnp.float32)
    mu = jnp.mean(x, axis=-1, keepdims=True)
    var = jnp.mean(jnp.square(x - mu), axis=-1, keepdims=True)
    xn = (x - mu) * jax.lax.rsqrt(var + 1e-5)
    xn = xn * g_ref[...].astype(jnp.float32) + beta_ref[...].astype(jnp.float32)
    xnb = xn.astype(jnp.bfloat16)

    qkv = jnp.dot(xnb, wqkv_ref[...],
                  preferred_element_type=jnp.float32).astype(jnp.bfloat16)
    q, k, v = (qkv[:, :inner], qkv[:, inner:2 * inner], qkv[:, 2 * inner:])

    h12 = jnp.dot(xnb, w1_ref[...],
                  preferred_element_type=jnp.float32) + b1_ref[...]
    hx, hg = h12[:, :ffd], h12[:, ffd:]
    sw = (hx * (hg * jax.nn.sigmoid(hg))).astype(jnp.bfloat16)
    acc = jnp.dot(sw, w2_ref[...],
                  preferred_element_type=jnp.float32) + b2_ref[...]

    dh = inner // heads
    for h in range(heads):
        sl = slice(h * dh, (h + 1) * dh)
        qh, kh, vh = q[:, sl], k[:, sl], v[:, sl]
        s = jax.lax.dot_general(qh, kh, (((1,), (1,)), ((), ())),
                                preferred_element_type=jnp.float32)
        m = jnp.max(s, axis=-1, keepdims=True)
        p = jnp.exp(s - m)
        inv_l = pl.reciprocal(jnp.sum(p, axis=-1, keepdims=True), approx=True)
        oh = jnp.dot(p.astype(jnp.bfloat16), vh,
                     preferred_element_type=jnp.float32) * inv_l
        acc = acc + jnp.dot(oh.astype(jnp.bfloat16), wo_ref[sl, :],
                            preferred_element_type=jnp.float32)

    o_ref[...] = acc.astype(o_ref.dtype)


def kernel(x, gamma, beta, wq, wkv, wo, w1, b1, w2, b2):
    B, N, D = x.shape
    heads = 4
    inner = wq.shape[1]
    ffd = w2.shape[0]
    scale = (inner // heads) ** -0.5
    bf = jnp.bfloat16

    wqkv = jnp.concatenate([wq * scale, wkv], axis=1).astype(bf)

    bmap = lambda b: (b, 0, 0)
    wmap = lambda b: (0, 0)
    full2 = lambda shape: pl.BlockSpec(shape, wmap)

    return pl.pallas_call(
        functools.partial(_fused_block_kernel, heads=heads, inner=inner,
                          ffd=ffd),
        out_shape=jax.ShapeDtypeStruct((B, N, D), x.dtype),
        grid_spec=pltpu.PrefetchScalarGridSpec(
            num_scalar_prefetch=0,
            grid=(B,),
            in_specs=[
                pl.BlockSpec((pl.Squeezed(), N, D), bmap),
                full2((1, D)), full2((1, D)),
                full2((D, 3 * inner)), full2((inner, D)),
                full2((D, 2 * ffd)), full2((1, 2 * ffd)),
                full2((ffd, D)), full2((1, D)),
            ],
            out_specs=pl.BlockSpec((pl.Squeezed(), N, D), bmap),
        ),
        compiler_params=pltpu.CompilerParams(
            dimension_semantics=("parallel",),
            vmem_limit_bytes=56 * 1024 * 1024),
    )(x, gamma, beta, wqkv, wo.astype(bf), w1.astype(bf), b1, w2.astype(bf),
      b2)
